# baseline (speedup 1.0000x reference)
import functools

import jax
import jax.numpy as jnp
from jax import lax
from jax.experimental import pallas as pl
from jax.experimental.pallas import tpu as pltpu
from jax.experimental.pallas import tpu_sc as plsc

_VOCAB_TILE = 1024


def _sc_info():
    try:
        info = plsc.get_sparse_core_info()
        return info.num_cores, info.num_subcores
    except Exception:
        return 2, 16


@functools.cache
def _make_gather_add(n, T, D):
    NC, NS = _sc_info()
    NW = NC * NS
    assert n % NW == 0 and (n // NW) % 8 == 0
    b_per_w = n // NW
    mesh = plsc.VectorSubcoreMesh(core_axis_name="c", subcore_axis_name="s")

    @functools.partial(
        pl.kernel,
        mesh=mesh,
        out_type=jax.ShapeDtypeStruct((n, D), jnp.float32),
        scratch_types=[
            pltpu.VMEM((b_per_w,), jnp.int32),
            pltpu.VMEM((b_per_w, D), jnp.float32),
            pltpu.VMEM((T, D), jnp.float32),
            pltpu.SemaphoreType.DMA,
        ],
        compiler_params=pltpu.CompilerParams(use_tc_tiling_on_sc=False),
    )
    def gather_add(idx_hbm, tok_hbm, pos_hbm, x_hbm, idx_v, rows_v, pos_v, sem):
        wid = lax.axis_index("s") * NC + lax.axis_index("c")
        base = wid * b_per_w
        pltpu.sync_copy(idx_hbm.at[pl.ds(base, b_per_w)], idx_v)
        pltpu.sync_copy(pos_hbm, pos_v)
        pltpu.async_copy(tok_hbm.at[idx_v], rows_v, sem).wait()
        for i in range(b_per_w):
            t = i % T
            for c in range(0, D, 16):
                rows_v[i, pl.ds(c, 16)] = (
                    rows_v[i, pl.ds(c, 16)] + pos_v[t, pl.ds(c, 16)]
                )
        pltpu.sync_copy(rows_v, x_hbm.at[pl.ds(base, b_per_w)])

    return gather_add


def _mm_body(x_ref, w_ref, b_ref, o_ref):
    o_ref[...] = (
        jnp.dot(x_ref[...], w_ref[...], preferred_element_type=jnp.float32)
        + b_ref[...]
    )


def _matmul_bias(x, W, b2d):
    n, d = x.shape
    V = W.shape[1]
    vt = _VOCAB_TILE
    return pl.pallas_call(
        _mm_body,
        grid=(pl.cdiv(V, vt),),
        in_specs=[
            pl.BlockSpec((n, d), lambda j: (0, 0)),
            pl.BlockSpec((d, vt), lambda j: (0, j)),
            pl.BlockSpec((1, vt), lambda j: (0, j)),
        ],
        out_specs=pl.BlockSpec((n, vt), lambda j: (0, j)),
        out_shape=jax.ShapeDtypeStruct((n, V), jnp.float32),
        compiler_params=pltpu.CompilerParams(
            dimension_semantics=("arbitrary",)
        ),
    )(x, W, b2d)


def kernel(idx, tok_table, pos_table, W, b):
    B, T = idx.shape
    V, D = tok_table.shape
    n = B * T
    x = _make_gather_add(n, T, D)(
        idx.reshape(n), tok_table, pos_table[:T].astype(jnp.float32)
    )
    logits = _matmul_bias(x, W, b.reshape(1, V))
    return logits.reshape(B, T, V)
